# baseline (speedup 1.0000x reference)
import functools

import jax
import jax.numpy as jnp
from jax import lax
from jax.experimental import pallas as pl
from jax.experimental.pallas import tpu as pltpu
from jax.experimental.pallas import tpu_sc as plsc

N_EMBD = 1024
B = 4
T = 4096
LANES = 16
NC = 2
NS = 16
NW = NC * NS
T_PER_W = T // NW
CHUNK = 32
N_CHUNK = T_PER_W // CHUNK
VECS_PER_ROW = N_EMBD // LANES


def kernel(idx, token_table, pos_table):
    idx_flat = idx.reshape(B * T)
    mesh = plsc.VectorSubcoreMesh(core_axis_name="c", subcore_axis_name="s")

    @functools.partial(
        pl.kernel,
        mesh=mesh,
        out_type=jax.ShapeDtypeStruct((B * T, N_EMBD), jnp.float32),
        scratch_types=[
            pltpu.VMEM((CHUNK,), jnp.int32),
            pltpu.VMEM((CHUNK, N_EMBD), jnp.float32),
            pltpu.VMEM((CHUNK, N_EMBD), jnp.float32),
            pltpu.SemaphoreType.DMA,
        ],
    )
    def emb_kernel(idx_hbm, tok_hbm, pos_hbm, out_hbm, idx_v, tok_v, pos_v, sem):
        wid = lax.axis_index("s") * NC + lax.axis_index("c")
        t_base = wid * T_PER_W
        for c in range(N_CHUNK):
            t0 = t_base + c * CHUNK
            pltpu.sync_copy(pos_hbm.at[pl.ds(t0, CHUNK)], pos_v)
            for b in range(B):
                pltpu.sync_copy(idx_hbm.at[pl.ds(b * T + t0, CHUNK)], idx_v)
                pltpu.async_copy(tok_hbm.at[idx_v], tok_v, sem).wait()

                def add_body(i, _):
                    r = i // VECS_PER_ROW
                    col = (i % VECS_PER_ROW) * LANES
                    plsc.addupdate(tok_v.at[r, pl.ds(col, LANES)],
                                   pos_v[r, pl.ds(col, LANES)])
                    return 0

                lax.fori_loop(0, CHUNK * VECS_PER_ROW, add_body, 0)
                pltpu.sync_copy(tok_v, out_hbm.at[pl.ds(b * T + t0, CHUNK)])

    out = emb_kernel(idx_flat, token_table, pos_table)
    return out.reshape(B, T, N_EMBD)


# speedup vs baseline: 2.0617x; 2.0617x over previous
import functools

import jax
import jax.numpy as jnp
from jax import lax
from jax.experimental import pallas as pl
from jax.experimental.pallas import tpu as pltpu
from jax.experimental.pallas import tpu_sc as plsc

N_EMBD = 1024
B = 4
T = 4096
LANES = 16
NC = 2
NS = 16
NW = NC * NS
T_PER_W = T // NW
CHUNK = 32
N_CHUNK = T_PER_W // CHUNK
N_ITEM = N_CHUNK * B
VECS_PER_ROW = N_EMBD // LANES


def kernel(idx, token_table, pos_table):
    idx_flat = idx.reshape(B * T)
    mesh = plsc.VectorSubcoreMesh(core_axis_name="c", subcore_axis_name="s")

    @functools.partial(
        pl.kernel,
        mesh=mesh,
        out_type=jax.ShapeDtypeStruct((B * T, N_EMBD), jnp.float32),
        scratch_types=[
            pltpu.VMEM((B, T_PER_W), jnp.int32),
            pltpu.VMEM((CHUNK, N_EMBD), jnp.float32),
            pltpu.VMEM((CHUNK, N_EMBD), jnp.float32),
            pltpu.VMEM((CHUNK, N_EMBD), jnp.float32),
            pltpu.SemaphoreType.DMA,
            pltpu.SemaphoreType.DMA,
            pltpu.SemaphoreType.DMA,
            pltpu.SemaphoreType.DMA,
            pltpu.SemaphoreType.DMA,
        ],
    )
    def emb_kernel(idx_hbm, tok_hbm, pos_hbm, out_hbm,
                   idx_v, tok0, tok1, pos_v,
                   gsem0, gsem1, osem0, osem1, psem):
        wid = lax.axis_index("s") * NC + lax.axis_index("c")
        t_base = wid * T_PER_W
        toks = (tok0, tok1)
        gsems = (gsem0, gsem1)
        osems = (osem0, osem1)

        for b in range(B):
            pltpu.sync_copy(idx_hbm.at[pl.ds(b * T + t_base, T_PER_W)],
                            idx_v.at[b])
        pltpu.sync_copy(pos_hbm.at[pl.ds(t_base, CHUNK)], pos_v)

        def gather(i, p):
            c, b = divmod(i, B)
            return pltpu.async_copy(
                tok_hbm.at[idx_v.at[b, pl.ds(c * CHUNK, CHUNK)]],
                toks[p], gsems[p])

        def out_copy(i, p):
            c, b = divmod(i, B)
            return pltpu.async_copy(
                toks[p], out_hbm.at[pl.ds(b * T + t_base + c * CHUNK, CHUNK)],
                osems[p])

        handles = {}
        handles["g0"] = gather(0, 0)
        pos_handle = None
        for i in range(N_ITEM):
            c, b = divmod(i, B)
            p = i % 2
            if i + 1 < N_ITEM:
                if i >= 1:
                    handles[f"o{i - 1}"].wait()
                handles[f"g{i + 1}"] = gather(i + 1, 1 - p)
            handles[f"g{i}"].wait()
            if b == 0 and pos_handle is not None:
                pos_handle.wait()
                pos_handle = None

            tok = toks[p]

            def add_body(r, _, tok=tok):
                for jv in range(VECS_PER_ROW):
                    col = jv * LANES
                    plsc.addupdate(tok.at[r, pl.ds(col, LANES)],
                                   pos_v[r, pl.ds(col, LANES)])
                return 0

            lax.fori_loop(0, CHUNK, add_body, 0)
            handles[f"o{i}"] = out_copy(i, p)
            if b == B - 1 and c + 1 < N_CHUNK:
                pos_handle = pltpu.async_copy(
                    pos_hbm.at[pl.ds(t_base + (c + 1) * CHUNK, CHUNK)],
                    pos_v, psem)
        handles[f"o{N_ITEM - 2}"].wait()
        handles[f"o{N_ITEM - 1}"].wait()

    out = emb_kernel(idx_flat, token_table, pos_table)
    return out.reshape(B, T, N_EMBD)


# speedup vs baseline: 2.5764x; 1.2497x over previous
import functools

import jax
import jax.numpy as jnp
from jax import lax
from jax.experimental import pallas as pl
from jax.experimental.pallas import tpu as pltpu
from jax.experimental.pallas import tpu_sc as plsc

N_EMBD = 1024
B = 4
T = 4096
LANES = 16
NC = 2
NS = 16
NW = NC * NS
T_PER_W = T // NW
CHUNK = 16
N_CHUNK = T_PER_W // CHUNK
N_ITEM = N_CHUNK * B
VECS_PER_ROW = N_EMBD // LANES
NBUF = 4


def kernel(idx, token_table, pos_table):
    mesh = plsc.VectorSubcoreMesh(core_axis_name="c", subcore_axis_name="s")

    @functools.partial(
        pl.kernel,
        mesh=mesh,
        out_type=jax.ShapeDtypeStruct((B * T, N_EMBD), jnp.float32),
        scratch_types=(
            [pltpu.VMEM((B, T_PER_W), jnp.int32)]
            + [pltpu.VMEM((CHUNK, N_EMBD), jnp.float32) for _ in range(NBUF)]
            + [pltpu.VMEM((CHUNK, N_EMBD), jnp.float32) for _ in range(2)]
            + [pltpu.SemaphoreType.DMA for _ in range(2 * NBUF + 2)]
        ),
    )
    def emb_kernel(idx_hbm, tok_hbm, pos_hbm, out_hbm, idx_v, *rest):
        toks = rest[:NBUF]
        poss = rest[NBUF:NBUF + 2]
        gsems = rest[NBUF + 2:2 * NBUF + 2]
        osems = rest[2 * NBUF + 2:3 * NBUF + 2]
        psems = rest[3 * NBUF + 2:]
        wid = lax.axis_index("s") * NC + lax.axis_index("c")
        t_base = wid * T_PER_W

        pltpu.sync_copy(idx_hbm.at[:, pl.ds(t_base, T_PER_W)], idx_v)

        def gather(i):
            c, b = divmod(i, B)
            p = i % NBUF
            return pltpu.async_copy(
                tok_hbm.at[idx_v.at[b, pl.ds(c * CHUNK, CHUNK)]],
                toks[p], gsems[p])

        def out_copy(i):
            c, b = divmod(i, B)
            p = i % NBUF
            return pltpu.async_copy(
                toks[p], out_hbm.at[pl.ds(b * T + t_base + c * CHUNK, CHUNK)],
                osems[p])

        def pos_load(c):
            return pltpu.async_copy(
                pos_hbm.at[pl.ds(t_base + c * CHUNK, CHUNK)],
                poss[c % 2], psems[c % 2])

        gh, oh, ph = {}, {}, {}
        ph[0] = pos_load(0)
        gh[0] = gather(0)
        gh[1] = gather(1)
        for i in range(N_ITEM):
            c, b = divmod(i, B)
            if i + 2 < N_ITEM:
                if i >= 2:
                    oh[i - 2].wait()
                gh[i + 2] = gather(i + 2)
            gh[i].wait()
            if b == 0:
                ph[c].wait()
                if c + 1 < N_CHUNK:
                    ph[c + 1] = pos_load(c + 1)
            tok = toks[i % NBUF]
            pos = poss[c % 2]

            def add_body(r, _, tok=tok, pos=pos):
                for jv in range(VECS_PER_ROW):
                    col = jv * LANES
                    plsc.addupdate(tok.at[r, pl.ds(col, LANES)],
                                   pos[r, pl.ds(col, LANES)])
                return 0

            lax.fori_loop(0, CHUNK, add_body, 0)
            oh[i] = out_copy(i)
        oh[N_ITEM - 4].wait()
        oh[N_ITEM - 3].wait()
        oh[N_ITEM - 2].wait()
        oh[N_ITEM - 1].wait()

    out = emb_kernel(idx, token_table, pos_table)
    return out.reshape(B, T, N_EMBD)
